# Initial kernel scaffold; baseline (speedup 1.0000x reference)
import jax
import jax.numpy as jnp
from jax.experimental import pallas as pl


def kernel(node_features, pred_len_t, input_len_t, edge_index_batchdata, edge_features_batchdata, W1, b1, W2, b2):
    raise NotImplementedError("write your pallas kernel here")


# speedup vs baseline: 18.3414x; 18.3414x over previous
import jax
import jax.numpy as jnp
from jax import lax
from jax.experimental import pallas as pl
from jax.experimental.pallas import tpu as pltpu
from jax.experimental.pallas import tpu_sc as plsc

B, N, D_FEAT = 4, 10000, 128
HID, OUT = 3, 4
NB = 10240
NP = B * NB
NR = B * N
E_B = 160000
TPB = 8
EPT = 20096
E_BP = TPB * EPT
NSTEP = EPT // 16

_MESH = plsc.VectorSubcoreMesh(core_axis_name="c", subcore_axis_name="s")
_CP = pltpu.CompilerParams(use_tc_tiling_on_sc=False,
                           needs_layout_passes=False)


def _make_sc_pass(C: int, gather: bool):
    def body(*refs):
        if gather:
            (g_hbm, src_hbm, dst_hbm, zeros_hbm, out_hbm,
             srcbuf, dstbuf, gtab, acc) = refs
        else:
            (dst_hbm, zeros_hbm, out_hbm, dstbuf, acc) = refs

        cid = lax.axis_index("c")
        sid = lax.axis_index("s")
        wid = cid * 16 + sid
        batch = wid // TPB

        pltpu.sync_copy(dst_hbm.at[wid], dstbuf)
        pltpu.sync_copy(zeros_hbm, acc)
        if gather:
            pltpu.sync_copy(src_hbm.at[wid], srcbuf)
            pltpu.sync_copy(g_hbm.at[batch], gtab)
        ones16 = jnp.full((16,), 1.0, jnp.float32)

        def step(i, carry):
            dv = dstbuf[pl.ds(i * 16, 16)] * C
            if gather:
                sv = srcbuf[pl.ds(i * 16, 16)] * C
                for c in range(C):
                    val = plsc.load_gather(gtab, [sv + c])
                    plsc.addupdate_scatter(acc, [dv + c], val)
            else:
                plsc.addupdate_scatter(acc, [dv], ones16)
            return carry

        lax.fori_loop(0, NSTEP, step, 0)
        pltpu.sync_copy(acc, out_hbm.at[wid])

    scratch = []
    if gather:
        scratch.append(pltpu.VMEM((EPT,), jnp.int32))
    scratch.append(pltpu.VMEM((EPT,), jnp.int32))
    if gather:
        scratch.append(pltpu.VMEM((NB * C,), jnp.float32))
    scratch.append(pltpu.VMEM((NB * C,), jnp.float32))

    return pl.kernel(
        body, mesh=_MESH,
        out_type=jax.ShapeDtypeStruct((32, NB * C), jnp.float32),
        scratch_types=scratch,
        compiler_params=_CP)


_sc_deg = _make_sc_pass(1, gather=False)
_sc_conv3 = _make_sc_pass(HID, gather=True)
_sc_conv4 = _make_sc_pass(OUT, gather=True)


_BLK = 256
_GRID = NP // _BLK
_BPB = NB // _BLK


def _tc1_body(x_ref, w1_ref, dp_ref, g1_ref, dinv_ref):
    cnt = jnp.sum(dp_ref[...], axis=0).reshape(_BLK, 1)
    dinv = lax.rsqrt(1.0 + cnt)
    h = jnp.dot(x_ref[...], w1_ref[...],
                preferred_element_type=jnp.float32)
    g1_ref[...] = h * dinv
    dinv_ref[...] = dinv


def _tc1(x, w1, deg_part):
    return pl.pallas_call(
        _tc1_body,
        grid=(_GRID,),
        in_specs=[
            pl.BlockSpec((_BLK, D_FEAT), lambda i: (i, 0)),
            pl.BlockSpec((D_FEAT, HID), lambda i: (0, 0)),
            pl.BlockSpec((TPB, _BLK), lambda i: (i // _BPB, i % _BPB)),
        ],
        out_specs=[
            pl.BlockSpec((_BLK, HID), lambda i: (i, 0)),
            pl.BlockSpec((_BLK, 1), lambda i: (i, 0)),
        ],
        out_shape=[
            jax.ShapeDtypeStruct((NP, HID), jnp.float32),
            jax.ShapeDtypeStruct((NP, 1), jnp.float32),
        ],
    )(x, w1, deg_part)


def _tc2_body(sp_ref, g1_ref, dinv_ref, w2_ref, b1_ref, g2_ref):
    i = pl.program_id(0)
    s1 = jnp.sum(sp_ref[0], axis=0)
    dinv = dinv_ref[...]
    pre = dinv * (g1_ref[...] + s1) + b1_ref[...]
    out1 = jnp.maximum(pre, 0.0)
    h2 = jnp.dot(out1, w2_ref[...], preferred_element_type=jnp.float32)
    local = (i % _BPB) * _BLK + lax.broadcasted_iota(jnp.int32, (_BLK, 1), 0)
    g2_ref[...] = jnp.where(local < N, h2 * dinv, 0.0)


def _tc2(s1_part, g1, dinv, w2, b1):
    return pl.pallas_call(
        _tc2_body,
        grid=(_GRID,),
        in_specs=[
            pl.BlockSpec((1, TPB, _BLK, HID),
                         lambda i: (i // _BPB, 0, i % _BPB, 0)),
            pl.BlockSpec((_BLK, HID), lambda i: (i, 0)),
            pl.BlockSpec((_BLK, 1), lambda i: (i, 0)),
            pl.BlockSpec((HID, OUT), lambda i: (0, 0)),
            pl.BlockSpec((1, HID), lambda i: (0, 0)),
        ],
        out_specs=pl.BlockSpec((_BLK, OUT), lambda i: (i, 0)),
        out_shape=jax.ShapeDtypeStruct((NP, OUT), jnp.float32),
    )(s1_part, g1, dinv, w2, b1)


def _tc3_body(sp_ref, g2_ref, dinv_ref, b2_ref, y_ref):
    s2 = jnp.sum(sp_ref[0], axis=0)
    y_ref[...] = dinv_ref[...] * (g2_ref[...] + s2) + b2_ref[...]


def _tc3(s2_part, g2, dinv, b2):
    return pl.pallas_call(
        _tc3_body,
        grid=(_GRID,),
        in_specs=[
            pl.BlockSpec((1, TPB, _BLK, OUT),
                         lambda i: (i // _BPB, 0, i % _BPB, 0)),
            pl.BlockSpec((_BLK, OUT), lambda i: (i, 0)),
            pl.BlockSpec((_BLK, 1), lambda i: (i, 0)),
            pl.BlockSpec((1, OUT), lambda i: (0, 0)),
        ],
        out_specs=pl.BlockSpec((_BLK, OUT), lambda i: (i, 0)),
        out_shape=jax.ShapeDtypeStruct((NP, OUT), jnp.float32),
    )(s2_part, g2, dinv, b2)


def kernel(node_features, pred_len_t, input_len_t, edge_index_batchdata,
           edge_features_batchdata, W1, b1, W2, b2):
    x = jnp.zeros((B, NB, D_FEAT), jnp.float32).at[:, :N].set(
        node_features.astype(jnp.float32)).reshape(NP, D_FEAT)

    pad = jnp.full((B, E_BP - E_B), N, jnp.int32)
    src = jnp.concatenate([edge_index_batchdata[:, 0, :], pad],
                          axis=1).reshape(32, EPT)
    dst = jnp.concatenate([edge_index_batchdata[:, 1, :], pad],
                          axis=1).reshape(32, EPT)

    z1 = jnp.zeros((NB,), jnp.float32)
    z3 = jnp.zeros((NB * HID,), jnp.float32)
    z4 = jnp.zeros((NB * OUT,), jnp.float32)

    deg_part = _sc_deg(dst, z1)
    g1, dinv = _tc1(x, W1, deg_part)
    s1_part = _sc_conv3(g1.reshape(B, NB * HID), src, dst, z3)
    g2 = _tc2(s1_part.reshape(B, TPB, NB, HID), g1, dinv,
              W2.astype(jnp.float32), b1.reshape(1, HID))
    s2_part = _sc_conv4(g2.reshape(B, NB * OUT), src, dst, z4)
    y = _tc3(s2_part.reshape(B, TPB, NB, OUT), g2, dinv,
             b2.reshape(1, OUT))
    return y.reshape(B, NB, OUT)[:, :N].reshape(NR, 1, OUT)


# speedup vs baseline: 75.3756x; 4.1096x over previous
import jax
import jax.numpy as jnp
from jax import lax
from jax.experimental import pallas as pl
from jax.experimental.pallas import tpu as pltpu
from jax.experimental.pallas import tpu_sc as plsc

B, N, D_FEAT = 4, 10000, 128
HID, OUT = 3, 4
NB = 10240
NP = B * NB
NR = B * N
E_B = 160000
TPB = 8
EPT = 20096
E_BP = TPB * EPT
NSTEP = EPT // 16

_MESH = plsc.VectorSubcoreMesh(core_axis_name="c", subcore_axis_name="s")
_CP = pltpu.CompilerParams(use_tc_tiling_on_sc=False,
                           needs_layout_passes=False)


def _sc_deg_body(dst_hbm, zeros_hbm, out_hbm, dstbuf, acc):
    cid = lax.axis_index("c")
    sid = lax.axis_index("s")
    wid = cid * 16 + sid

    pltpu.sync_copy(dst_hbm.at[wid], dstbuf)
    pltpu.sync_copy(zeros_hbm, acc)
    ones16 = jnp.full((16,), 1.0, jnp.float32)

    def step(i, carry):
        dv = dstbuf[pl.ds(i * 16, 16)]
        plsc.addupdate_scatter(acc, [dv], ones16)
        return carry

    lax.fori_loop(0, NSTEP, step, 0)
    pltpu.sync_copy(acc, out_hbm.at[wid])


_sc_deg = pl.kernel(
    _sc_deg_body, mesh=_MESH,
    out_type=jax.ShapeDtypeStruct((32, NB), jnp.float32),
    scratch_types=[pltpu.VMEM((EPT,), jnp.int32),
                   pltpu.VMEM((NB,), jnp.float32)],
    compiler_params=_CP)


def _make_sc_conv(C: int):
    def body(g_hbm, src_hbm, dst_hbm, zeros_hbm, out_hbm,
             srcbuf, dstbuf, gtab, acc):
        cid = lax.axis_index("c")
        sid = lax.axis_index("s")
        wid = cid * 16 + sid
        batch = wid // TPB

        pltpu.sync_copy(dst_hbm.at[wid], dstbuf)
        pltpu.sync_copy(src_hbm.at[wid], srcbuf)
        pltpu.sync_copy(zeros_hbm, acc)
        pltpu.sync_copy(g_hbm.at[:, pl.ds(batch * NB, NB)], gtab)
        cvec = [jnp.full((16,), c, jnp.int32) for c in range(C)]

        def step(i, carry):
            sv = srcbuf[pl.ds(i * 16, 16)]
            dv = dstbuf[pl.ds(i * 16, 16)]
            for c in range(C):
                val = plsc.load_gather(gtab, [cvec[c], sv])
                plsc.addupdate_scatter(acc, [cvec[c], dv], val)
            return carry

        lax.fori_loop(0, NSTEP, step, 0)
        pltpu.sync_copy(acc, out_hbm.at[wid])

    return pl.kernel(
        body, mesh=_MESH,
        out_type=jax.ShapeDtypeStruct((32, C, NB), jnp.float32),
        scratch_types=[pltpu.VMEM((EPT,), jnp.int32),
                       pltpu.VMEM((EPT,), jnp.int32),
                       pltpu.VMEM((C, NB), jnp.float32),
                       pltpu.VMEM((C, NB), jnp.float32)],
        compiler_params=_CP)


_sc_conv3 = _make_sc_conv(HID)
_sc_conv4 = _make_sc_conv(OUT)


_BLK = 2048
_GRID = NP // _BLK
_BPB = NB // _BLK


def _tc1_body(xt_ref, w1t_ref, dp_ref, g1_ref, dinv_ref):
    cnt = jnp.sum(dp_ref[...], axis=0).reshape(1, _BLK)
    dinv = lax.rsqrt(1.0 + cnt)
    h = jnp.dot(w1t_ref[...], xt_ref[...],
                preferred_element_type=jnp.float32)
    g1_ref[...] = h * dinv
    dinv_ref[...] = dinv


def _tc1(xt, w1t, deg_part):
    return pl.pallas_call(
        _tc1_body,
        grid=(_GRID,),
        in_specs=[
            pl.BlockSpec((D_FEAT, _BLK), lambda i: (0, i)),
            pl.BlockSpec((HID, D_FEAT), lambda i: (0, 0)),
            pl.BlockSpec((TPB, _BLK), lambda i: (i // _BPB, i % _BPB)),
        ],
        out_specs=[
            pl.BlockSpec((HID, _BLK), lambda i: (0, i)),
            pl.BlockSpec((1, _BLK), lambda i: (0, i)),
        ],
        out_shape=[
            jax.ShapeDtypeStruct((HID, NP), jnp.float32),
            jax.ShapeDtypeStruct((1, NP), jnp.float32),
        ],
    )(xt, w1t, deg_part)


def _tc2_body(sp_ref, g1_ref, dinv_ref, w2t_ref, b1_ref, g2_ref):
    i = pl.program_id(0)
    s1 = jnp.sum(sp_ref[...], axis=0)
    dinv = dinv_ref[...]
    pre = dinv * (g1_ref[...] + s1) + b1_ref[...]
    out1 = jnp.maximum(pre, 0.0)
    h2 = jnp.dot(w2t_ref[...], out1,
                 preferred_element_type=jnp.float32)
    local = (i % _BPB) * _BLK + lax.broadcasted_iota(jnp.int32, (1, _BLK), 1)
    g2_ref[...] = jnp.where(local < N, h2 * dinv, 0.0)


def _tc2(s1_part, g1, dinv, w2t, b1):
    return pl.pallas_call(
        _tc2_body,
        grid=(_GRID,),
        in_specs=[
            pl.BlockSpec((TPB, HID, _BLK), lambda i: (i // _BPB, 0, i % _BPB)),
            pl.BlockSpec((HID, _BLK), lambda i: (0, i)),
            pl.BlockSpec((1, _BLK), lambda i: (0, i)),
            pl.BlockSpec((OUT, HID), lambda i: (0, 0)),
            pl.BlockSpec((HID, 1), lambda i: (0, 0)),
        ],
        out_specs=pl.BlockSpec((OUT, _BLK), lambda i: (0, i)),
        out_shape=jax.ShapeDtypeStruct((OUT, NP), jnp.float32),
    )(s1_part, g1, dinv, w2t, b1)


def _tc3_body(sp_ref, g2_ref, dinv_ref, b2_ref, y_ref):
    s2 = jnp.sum(sp_ref[...], axis=0)
    y_ref[...] = dinv_ref[...] * (g2_ref[...] + s2) + b2_ref[...]


def _tc3(s2_part, g2, dinv, b2):
    return pl.pallas_call(
        _tc3_body,
        grid=(_GRID,),
        in_specs=[
            pl.BlockSpec((TPB, OUT, _BLK), lambda i: (i // _BPB, 0, i % _BPB)),
            pl.BlockSpec((OUT, _BLK), lambda i: (0, i)),
            pl.BlockSpec((1, _BLK), lambda i: (0, i)),
            pl.BlockSpec((OUT, 1), lambda i: (0, 0)),
        ],
        out_specs=pl.BlockSpec((OUT, _BLK), lambda i: (0, i)),
        out_shape=jax.ShapeDtypeStruct((OUT, NP), jnp.float32),
    )(s2_part, g2, dinv, b2)


def kernel(node_features, pred_len_t, input_len_t, edge_index_batchdata,
           edge_features_batchdata, W1, b1, W2, b2):
    xt = jnp.zeros((D_FEAT, B, NB), jnp.float32).at[:, :, :N].set(
        node_features.astype(jnp.float32).transpose(2, 0, 1)
    ).reshape(D_FEAT, NP)

    pad = jnp.full((B, E_BP - E_B), N, jnp.int32)
    src = jnp.concatenate([edge_index_batchdata[:, 0, :], pad],
                          axis=1).reshape(32, EPT)
    dst = jnp.concatenate([edge_index_batchdata[:, 1, :], pad],
                          axis=1).reshape(32, EPT)

    z1 = jnp.zeros((NB,), jnp.float32)
    z3 = jnp.zeros((HID, NB), jnp.float32)
    z4 = jnp.zeros((OUT, NB), jnp.float32)
    w1t = W1.astype(jnp.float32).T
    w2t = W2.astype(jnp.float32).T

    deg_part = _sc_deg(dst, z1)
    g1, dinv = _tc1(xt, w1t, deg_part)
    s1_part = _sc_conv3(g1, src, dst, z3)
    g2 = _tc2(s1_part, g1, dinv, w2t, b1.reshape(HID, 1))
    s2_part = _sc_conv4(g2, src, dst, z4)
    y = _tc3(s2_part, g2, dinv, b2.reshape(OUT, 1))
    y = y.reshape(OUT, B, NB)[:, :, :N].transpose(1, 2, 0)
    return y.reshape(NR, 1, OUT)


# speedup vs baseline: 82.1746x; 1.0902x over previous
import jax
import jax.numpy as jnp
from jax import lax
from jax.experimental import pallas as pl
from jax.experimental.pallas import tpu as pltpu
from jax.experimental.pallas import tpu_sc as plsc

B, N, D_FEAT = 4, 10000, 128
HID, OUT = 3, 4
NB = 10240
NP = B * NB
NR = B * N
E_B = 160000
TPB = 8
EPT = 20096
E_BP = TPB * EPT
NSTEP = EPT // 16
UNROLL = 4
NMAIN = NSTEP // UNROLL * UNROLL

_MESH = plsc.VectorSubcoreMesh(core_axis_name="c", subcore_axis_name="s")
_CP = pltpu.CompilerParams(use_tc_tiling_on_sc=False,
                           needs_layout_passes=False)


def _sc_deg_body(dst_hbm, zeros_hbm, out_hbm, dstbuf, acc):
    cid = lax.axis_index("c")
    sid = lax.axis_index("s")
    wid = cid * 16 + sid

    pltpu.sync_copy(dst_hbm.at[wid], dstbuf)
    pltpu.sync_copy(zeros_hbm, acc)
    ones16 = jnp.full((16,), 1.0, jnp.float32)

    def step(i, carry):
        for u in range(UNROLL):
            dv = dstbuf[pl.ds(i * (16 * UNROLL) + u * 16, 16)]
            plsc.addupdate_scatter(acc, [dv], ones16)
        return carry

    lax.fori_loop(0, NSTEP // UNROLL, step, 0)
    for j in range(NMAIN, NSTEP):
        dv = dstbuf[pl.ds(j * 16, 16)]
        plsc.addupdate_scatter(acc, [dv], ones16)
    pltpu.sync_copy(acc, out_hbm.at[wid])


_sc_deg = pl.kernel(
    _sc_deg_body, mesh=_MESH,
    out_type=jax.ShapeDtypeStruct((32, NB), jnp.float32),
    scratch_types=[pltpu.VMEM((EPT,), jnp.int32),
                   pltpu.VMEM((NB,), jnp.float32)],
    compiler_params=_CP)


def _make_sc_conv(C: int):
    def body(g_hbm, src_hbm, dst_hbm, zeros_hbm, out_hbm,
             srcbuf, dstbuf, gtab, acc):
        cid = lax.axis_index("c")
        sid = lax.axis_index("s")
        wid = cid * 16 + sid
        batch = wid // TPB

        pltpu.sync_copy(dst_hbm.at[wid], dstbuf)
        pltpu.sync_copy(src_hbm.at[wid], srcbuf)
        pltpu.sync_copy(zeros_hbm, acc)
        pltpu.sync_copy(g_hbm.at[:, pl.ds(batch * NB, NB)], gtab)
        cvec = [jnp.full((16,), c, jnp.int32) for c in range(C)]

        def one(j):
            sv = srcbuf[pl.ds(j, 16)]
            dv = dstbuf[pl.ds(j, 16)]
            for c in range(C):
                val = plsc.load_gather(gtab, [cvec[c], sv])
                plsc.addupdate_scatter(acc, [cvec[c], dv], val)

        def step(i, carry):
            for u in range(UNROLL):
                one(i * (16 * UNROLL) + u * 16)
            return carry

        lax.fori_loop(0, NSTEP // UNROLL, step, 0)
        for j in range(NMAIN, NSTEP):
            one(j * 16)
        pltpu.sync_copy(acc, out_hbm.at[wid])

    return pl.kernel(
        body, mesh=_MESH,
        out_type=jax.ShapeDtypeStruct((32, C, NB), jnp.float32),
        scratch_types=[pltpu.VMEM((EPT,), jnp.int32),
                       pltpu.VMEM((EPT,), jnp.int32),
                       pltpu.VMEM((C, NB), jnp.float32),
                       pltpu.VMEM((C, NB), jnp.float32)],
        compiler_params=_CP)


_sc_conv3 = _make_sc_conv(HID)
_sc_conv4 = _make_sc_conv(OUT)


_BLK = 2048
_GRID = NP // _BLK
_BPB = NB // _BLK

_DN = (((1,), (1,)), ((), ()))


def _tc1_body(x_ref, w1t_ref, dp_ref, g1_ref, dinv_ref):
    cnt = jnp.sum(dp_ref[...], axis=0).reshape(1, _BLK)
    dinv = lax.rsqrt(1.0 + cnt)
    h = lax.dot_general(w1t_ref[...], x_ref[...], _DN,
                        preferred_element_type=jnp.float32)
    g1_ref[...] = h * dinv
    dinv_ref[...] = dinv


def _tc1(x, w1t, deg_part):
    return pl.pallas_call(
        _tc1_body,
        grid=(_GRID,),
        in_specs=[
            pl.BlockSpec((_BLK, D_FEAT), lambda i: (i, 0)),
            pl.BlockSpec((HID, D_FEAT), lambda i: (0, 0)),
            pl.BlockSpec((TPB, _BLK), lambda i: (i // _BPB, i % _BPB)),
        ],
        out_specs=[
            pl.BlockSpec((HID, _BLK), lambda i: (0, i)),
            pl.BlockSpec((1, _BLK), lambda i: (0, i)),
        ],
        out_shape=[
            jax.ShapeDtypeStruct((HID, NP), jnp.float32),
            jax.ShapeDtypeStruct((1, NP), jnp.float32),
        ],
    )(x, w1t, deg_part)


def _tc2_body(sp_ref, g1_ref, dinv_ref, w2t_ref, b1_ref, g2_ref):
    i = pl.program_id(0)
    s1 = jnp.sum(sp_ref[...], axis=0)
    dinv = dinv_ref[...]
    pre = dinv * (g1_ref[...] + s1) + b1_ref[...]
    out1 = jnp.maximum(pre, 0.0)
    h2 = jnp.dot(w2t_ref[...], out1,
                 preferred_element_type=jnp.float32)
    local = (i % _BPB) * _BLK + lax.broadcasted_iota(jnp.int32, (1, _BLK), 1)
    g2_ref[...] = jnp.where(local < N, h2 * dinv, 0.0)


def _tc2(s1_part, g1, dinv, w2t, b1):
    return pl.pallas_call(
        _tc2_body,
        grid=(_GRID,),
        in_specs=[
            pl.BlockSpec((TPB, HID, _BLK), lambda i: (i // _BPB, 0, i % _BPB)),
            pl.BlockSpec((HID, _BLK), lambda i: (0, i)),
            pl.BlockSpec((1, _BLK), lambda i: (0, i)),
            pl.BlockSpec((OUT, HID), lambda i: (0, 0)),
            pl.BlockSpec((HID, 1), lambda i: (0, 0)),
        ],
        out_specs=pl.BlockSpec((OUT, _BLK), lambda i: (0, i)),
        out_shape=jax.ShapeDtypeStruct((OUT, NP), jnp.float32),
    )(s1_part, g1, dinv, w2t, b1)


def _tc3_body(sp_ref, g2_ref, dinv_ref, b2_ref, y_ref):
    s2 = jnp.sum(sp_ref[...], axis=0)
    y_ref[...] = dinv_ref[...] * (g2_ref[...] + s2) + b2_ref[...]


def _tc3(s2_part, g2, dinv, b2):
    return pl.pallas_call(
        _tc3_body,
        grid=(_GRID,),
        in_specs=[
            pl.BlockSpec((TPB, OUT, _BLK), lambda i: (i // _BPB, 0, i % _BPB)),
            pl.BlockSpec((OUT, _BLK), lambda i: (0, i)),
            pl.BlockSpec((1, _BLK), lambda i: (0, i)),
            pl.BlockSpec((OUT, 1), lambda i: (0, 0)),
        ],
        out_specs=pl.BlockSpec((OUT, _BLK), lambda i: (0, i)),
        out_shape=jax.ShapeDtypeStruct((OUT, NP), jnp.float32),
    )(s2_part, g2, dinv, b2)


def kernel(node_features, pred_len_t, input_len_t, edge_index_batchdata,
           edge_features_batchdata, W1, b1, W2, b2):
    x = jnp.zeros((B, NB, D_FEAT), jnp.float32).at[:, :N].set(
        node_features.astype(jnp.float32)).reshape(NP, D_FEAT)

    pad = jnp.full((B, E_BP - E_B), N, jnp.int32)
    src = jnp.concatenate([edge_index_batchdata[:, 0, :], pad],
                          axis=1).reshape(32, EPT)
    dst = jnp.concatenate([edge_index_batchdata[:, 1, :], pad],
                          axis=1).reshape(32, EPT)

    z1 = jnp.zeros((NB,), jnp.float32)
    z3 = jnp.zeros((HID, NB), jnp.float32)
    z4 = jnp.zeros((OUT, NB), jnp.float32)
    w1t = W1.astype(jnp.float32).T
    w2t = W2.astype(jnp.float32).T

    deg_part = _sc_deg(dst, z1)
    g1, dinv = _tc1(x, w1t, deg_part)
    s1_part = _sc_conv3(g1, src, dst, z3)
    g2 = _tc2(s1_part, g1, dinv, w2t, b1.reshape(HID, 1))
    s2_part = _sc_conv4(g2, src, dst, z4)
    y = _tc3(s2_part, g2, dinv, b2.reshape(OUT, 1))
    y = y.reshape(OUT, B, NB)[:, :, :N].transpose(1, 2, 0)
    return y.reshape(NR, 1, OUT)
